# Initial kernel scaffold; baseline (speedup 1.0000x reference)
import jax
import jax.numpy as jnp
from jax import lax
from jax.experimental import pallas as pl
from jax.experimental.pallas import tpu as pltpu
from jax.experimental.pallas import tpu_sc as plsc

USER = 5000
ITEM = 5000
LATDIM = 128
HALF = 64
N_EDGES = 320000
UU_EDGES = 160000
NC = 2
NS = 16
CH = 80

UI_NCH = N_EDGES // NS // CH
UU_NCH = UU_EDGES // NS // CH
UI_R = 10240
UU_R = 5120
UI_RP = UI_R // NS
UU_RP = UU_R // NS


def _sc_kernel(ui_t0, uu_t0, ui_col, ui_row, ui_val, uu_col, uu_row, uu_val,
               zeros_hbm,
               cur1, cur2, cur3, ucur1, ucur2,
               colv, rowv, valv, gbuf):
  cid = lax.axis_index("c")
  sid = lax.axis_index("s")

  def scale_chunk(j):
    @pl.loop(0, CH)
    def _(e):
      v = jnp.full((16,), valv[j, e], dtype=jnp.float32)
      for c in range(HALF // 16):
        sl = (e, pl.ds(c * 16, 16))
        gbuf[sl] = gbuf[sl] * v

  def layer(t_src, out_hbm, ospm, nch, rp):
    @pl.loop(0, nch)
    def _(j):
      pltpu.sync_copy(t_src.at[cid].at[colv.at[j]], gbuf)
      scale_chunk(j)
      pltpu.sync_copy(gbuf, ospm.at[rowv.at[j]], add=True)
    plsc.subcore_barrier()
    base = sid * rp
    rows = pl.ds(base, rp)
    pltpu.sync_copy(ospm.at[rows], out_hbm.at[cid].at[rows])
    pltpu.sync_copy(zeros_hbm.at[rows], ospm.at[rows])
    plsc.subcore_barrier()

  def run_graph(t0, col_h, row_h, val_h, outs, nch, rp, ospm):
    nsl = pl.ds(0, nch)
    pltpu.sync_copy(col_h.at[sid], colv.at[nsl])
    pltpu.sync_copy(row_h.at[sid], rowv.at[nsl])
    pltpu.sync_copy(val_h.at[sid], valv.at[nsl])
    src = t0
    for out_hbm in outs:
      layer(src, out_hbm, ospm, nch, rp)
      src = out_hbm

  def body(ospm):
    rows = pl.ds(sid * UI_RP, UI_RP)
    pltpu.sync_copy(zeros_hbm.at[rows], ospm.at[rows])
    plsc.subcore_barrier()
    run_graph(ui_t0, ui_col, ui_row, ui_val, (cur1, cur2, cur3),
              UI_NCH, UI_RP, ospm)
    run_graph(uu_t0, uu_col, uu_row, uu_val, (ucur1, ucur2),
              UU_NCH, UU_RP, ospm)

  pl.run_scoped(body, pltpu.VMEM_SHARED((UI_R, HALF), jnp.float32))


def _combine_body(uE, iE, c1, c2, c3, u1, u2, ou, oi, ouu):
  s = c1[...] + c2[...] + c3[...]
  full = jnp.concatenate([s[0], s[1]], axis=1)
  ou[...] = uE[...] + full[:USER]
  oi[...] = iE[...] + full[USER:USER + ITEM]
  su = u1[...] + u2[...]
  fu = jnp.concatenate([su[0], su[1]], axis=1)
  ouu[...] = uE[...] + fu[:USER]


@jax.jit
def kernel(adj_indices, adj_values, uadj_indices, uadj_values, uEmbeds0,
           iEmbeds0):
  f32 = jnp.float32
  embeds = jnp.concatenate([uEmbeds0, iEmbeds0], axis=0)
  emb_p = jnp.pad(embeds, ((0, UI_R - USER - ITEM), (0, 0)))
  ui_t0 = jnp.stack([emb_p[:, :HALF], emb_p[:, HALF:]])
  ue_p = jnp.pad(uEmbeds0, ((0, UU_R - USER), (0, 0)))
  uu_t0 = jnp.stack([ue_p[:, :HALF], ue_p[:, HALF:]])

  ui_col = adj_indices[1].astype(jnp.int32).reshape(NS, UI_NCH, CH)
  ui_row = adj_indices[0].astype(jnp.int32).reshape(NS, UI_NCH, CH)
  ui_val = adj_values.reshape(NS, UI_NCH, CH)
  uu_col = uadj_indices[1].astype(jnp.int32).reshape(NS, UU_NCH, CH)
  uu_row = uadj_indices[0].astype(jnp.int32).reshape(NS, UU_NCH, CH)
  uu_val = uadj_values.reshape(NS, UU_NCH, CH)
  zeros_hbm = jnp.zeros((UI_R, HALF), f32)

  sds = jax.ShapeDtypeStruct
  sc = pl.kernel(
      _sc_kernel,
      out_type=[
          sds((NC, UI_R, HALF), f32),
          sds((NC, UI_R, HALF), f32),
          sds((NC, UI_R, HALF), f32),
          sds((NC, UU_R, HALF), f32),
          sds((NC, UU_R, HALF), f32),
      ],
      mesh=plsc.VectorSubcoreMesh(core_axis_name="c", subcore_axis_name="s"),
      scratch_types=[
          pltpu.VMEM((UI_NCH, CH), jnp.int32),
          pltpu.VMEM((UI_NCH, CH), jnp.int32),
          pltpu.VMEM((UI_NCH, CH), f32),
          pltpu.VMEM((CH, HALF), f32),
      ],
  )
  cur1, cur2, cur3, ucur1, ucur2 = sc(
      ui_t0, uu_t0, ui_col, ui_row, ui_val, uu_col, uu_row, uu_val, zeros_hbm)

  ou, oi, ouu = pl.pallas_call(
      _combine_body,
      out_shape=[
          sds((USER, LATDIM), f32),
          sds((ITEM, LATDIM), f32),
          sds((USER, LATDIM), f32),
      ],
  )(uEmbeds0, iEmbeds0, cur1, cur2, cur3, ucur1, ucur2)
  return ou, oi, ouu


# speedup vs baseline: 2.4087x; 2.4087x over previous
import jax
import jax.numpy as jnp
from jax import lax
from jax.experimental import pallas as pl
from jax.experimental.pallas import tpu as pltpu
from jax.experimental.pallas import tpu_sc as plsc

USER = 5000
ITEM = 5000
LATDIM = 128
HALF = 64
N_EDGES = 320000
UU_EDGES = 160000
NC = 2
NS = 16
CH = 80

UI_NCH = N_EDGES // NS // CH
UU_NCH = UU_EDGES // NS // CH
UI_R = 10240
UU_R = 5120
UI_RP = UI_R // NS
UU_RP = UU_R // NS


def _sc_kernel(ui_t0, uu_t0, ui_col, ui_row, ui_val, uu_col, uu_row, uu_val,
               zeros_hbm,
               cur1, cur2, cur3, ucur1, ucur2,
               colv, rowv, valv, gbuf, ospm):
  cid = lax.axis_index("c")
  sid = lax.axis_index("s")

  def scale_chunk(j):
    @pl.loop(0, CH, step=16)
    def _(g):
      vv = valv[j, pl.ds(g, 16)]
      for e16 in range(16):
        bv = jnp.full((16,), vv[e16], dtype=jnp.float32)
        for c in range(HALF // 16):
          sl = (g + e16, pl.ds(c * 16, 16))
          gbuf[sl] = gbuf[sl] * bv

  def layer(t_src, out_hbm, ospm, nch, rp):
    @pl.loop(0, nch)
    def _(j):
      pltpu.sync_copy(t_src.at[cid].at[colv.at[j]], gbuf)
      scale_chunk(j)
      pltpu.sync_copy(gbuf, ospm.at[rowv.at[j]], add=True)
    plsc.subcore_barrier()
    base = sid * rp
    rows = pl.ds(base, rp)
    pltpu.sync_copy(ospm.at[rows], out_hbm.at[cid].at[rows])
    pltpu.sync_copy(zeros_hbm.at[rows], ospm.at[rows])
    plsc.subcore_barrier()

  def run_graph(t0, col_h, row_h, val_h, outs, nch, rp, ospm):
    nsl = pl.ds(0, nch)
    pltpu.sync_copy(col_h.at[sid], colv.at[nsl])
    pltpu.sync_copy(row_h.at[sid], rowv.at[nsl])
    pltpu.sync_copy(val_h.at[sid], valv.at[nsl])
    src = t0
    for out_hbm in outs:
      layer(src, out_hbm, ospm, nch, rp)
      src = out_hbm

  rows0 = pl.ds(sid * UI_RP, UI_RP)
  pltpu.sync_copy(zeros_hbm.at[rows0], ospm.at[rows0])
  plsc.subcore_barrier()
  run_graph(ui_t0, ui_col, ui_row, ui_val, (cur1, cur2, cur3),
            UI_NCH, UI_RP, ospm)
  run_graph(uu_t0, uu_col, uu_row, uu_val, (ucur1, ucur2),
            UU_NCH, UU_RP, ospm)


def _combine_body(uE, iE, c1, c2, c3, u1, u2, ou, oi, ouu):
  s = c1[...] + c2[...] + c3[...]
  full = jnp.concatenate([s[0], s[1]], axis=1)
  ou[...] = uE[...] + full[:USER]
  oi[...] = iE[...] + full[USER:USER + ITEM]
  su = u1[...] + u2[...]
  fu = jnp.concatenate([su[0], su[1]], axis=1)
  ouu[...] = uE[...] + fu[:USER]


@jax.jit
def kernel(adj_indices, adj_values, uadj_indices, uadj_values, uEmbeds0,
           iEmbeds0):
  f32 = jnp.float32
  embeds = jnp.concatenate([uEmbeds0, iEmbeds0], axis=0)
  emb_p = jnp.pad(embeds, ((0, UI_R - USER - ITEM), (0, 0)))
  ui_t0 = jnp.stack([emb_p[:, :HALF], emb_p[:, HALF:]])
  ue_p = jnp.pad(uEmbeds0, ((0, UU_R - USER), (0, 0)))
  uu_t0 = jnp.stack([ue_p[:, :HALF], ue_p[:, HALF:]])

  ui_col = adj_indices[1].astype(jnp.int32).reshape(NS, UI_NCH, CH)
  ui_row = adj_indices[0].astype(jnp.int32).reshape(NS, UI_NCH, CH)
  ui_val = adj_values.reshape(NS, UI_NCH, CH)
  uu_col = uadj_indices[1].astype(jnp.int32).reshape(NS, UU_NCH, CH)
  uu_row = uadj_indices[0].astype(jnp.int32).reshape(NS, UU_NCH, CH)
  uu_val = uadj_values.reshape(NS, UU_NCH, CH)
  zeros_hbm = jnp.zeros((UI_R, HALF), f32)

  sds = jax.ShapeDtypeStruct
  sc = pl.kernel(
      _sc_kernel,
      out_type=[
          sds((NC, UI_R, HALF), f32),
          sds((NC, UI_R, HALF), f32),
          sds((NC, UI_R, HALF), f32),
          sds((NC, UU_R, HALF), f32),
          sds((NC, UU_R, HALF), f32),
      ],
      mesh=plsc.VectorSubcoreMesh(core_axis_name="c", subcore_axis_name="s"),
      compiler_params=pltpu.CompilerParams(use_tc_tiling_on_sc=False),
      scratch_types=[
          pltpu.VMEM((UI_NCH, CH), jnp.int32),
          pltpu.VMEM((UI_NCH, CH), jnp.int32),
          pltpu.VMEM((UI_NCH, CH), f32),
          pltpu.VMEM((CH, HALF), f32),
          pltpu.VMEM_SHARED((UI_R, HALF), f32),
      ],
  )
  cur1, cur2, cur3, ucur1, ucur2 = sc(
      ui_t0, uu_t0, ui_col, ui_row, ui_val, uu_col, uu_row, uu_val, zeros_hbm)

  ou, oi, ouu = pl.pallas_call(
      _combine_body,
      out_shape=[
          sds((USER, LATDIM), f32),
          sds((ITEM, LATDIM), f32),
          sds((USER, LATDIM), f32),
      ],
  )(uEmbeds0, iEmbeds0, cur1, cur2, cur3, ucur1, ucur2)
  return ou, oi, ouu


# speedup vs baseline: 3.6487x; 1.5148x over previous
import jax
import jax.numpy as jnp
from jax import lax
from jax.experimental import pallas as pl
from jax.experimental.pallas import tpu as pltpu
from jax.experimental.pallas import tpu_sc as plsc

USER = 5000
ITEM = 5000
LATDIM = 128
HALF = 64
N_EDGES = 320000
UU_EDGES = 160000
NC = 2
NS = 16
CH = 80
NB = 4
NBI = 3 * NB

UI_NCH = N_EDGES // NS // CH
UU_NCH = UU_EDGES // NS // CH
UI_R = 10240
UU_R = 5120
UI_RP = UI_R // NS
UU_RP = UU_R // NS


def _sc_kernel(ui_t0, uu_t0, ui_col, ui_row, ui_val, uu_col, uu_row, uu_val,
               zeros_hbm,
               cur1, cur2, cur3, ucur1, ucur2,
               colring, rowv, valv, gbuf, sbuf, ospm, gsem, ssem, isem):
  cid = lax.axis_index("c")
  sid = lax.axis_index("s")

  def scale_chunk(j, b):
    @pl.loop(0, CH, step=16)
    def _(g):
      vv = valv[j, pl.ds(g, 16)]
      for e16 in range(16):
        bv = jnp.full((16,), vv[e16], dtype=jnp.float32)
        for c in range(HALF // 16):
          sl = (b, g + e16, pl.ds(c * 16, 16))
          sbuf[sl] = gbuf[sl] * bv

  def layer(t_src, col_h, out_hbm, ospm, nch, rp):
    src = t_src.at[cid]
    colh = col_h.at[sid]

    def idx_dma(j):
      s = lax.rem(j, NBI)
      pltpu.async_copy(colh.at[j], colring.at[s], isem.at[s])

    def idx_wait(j):
      s = lax.rem(j, NBI)
      pltpu.make_async_copy(colh.at[j], colring.at[s], isem.at[s]).wait()

    def gather_start(j):
      b, s = lax.rem(j, NB), lax.rem(j, NBI)
      pltpu.async_copy(src.at[colring.at[s]], gbuf.at[b], gsem.at[b])

    def gather_wait(j):
      b, s = lax.rem(j, NB), lax.rem(j, NBI)
      pltpu.make_async_copy(src.at[colring.at[s]], gbuf.at[b],
                            gsem.at[b]).wait()

    @pl.loop(0, 2 * NB)
    def _(j):
      idx_dma(j)

    @pl.loop(0, NB)
    def _(j):
      idx_wait(j)
      gather_start(j)

    @pl.loop(0, nch)
    def _(j):
      b = lax.rem(j, NB)
      gather_wait(j)

      @pl.when(j >= NB)
      def _():
        pltpu.make_async_copy(sbuf.at[b], ospm.at[rowv.at[j]],
                              ssem.at[b]).wait()

      scale_chunk(j, b)

      @pl.when(j + NB < nch)
      def _():
        idx_wait(j + NB)
        gather_start(j + NB)

      @pl.when(j + 2 * NB < nch)
      def _():
        idx_dma(j + 2 * NB)

      pltpu.async_copy(sbuf.at[b], ospm.at[rowv.at[j]], ssem.at[b], add=True)

    @pl.loop(nch - NB, nch)
    def _(j):
      b = lax.rem(j, NB)
      pltpu.make_async_copy(sbuf.at[b], ospm.at[rowv.at[j]], ssem.at[b]).wait()

    plsc.subcore_barrier()
    base = sid * rp
    rows = pl.ds(base, rp)
    pltpu.sync_copy(ospm.at[rows], out_hbm.at[cid].at[rows])
    pltpu.sync_copy(zeros_hbm.at[rows], ospm.at[rows])
    plsc.subcore_barrier()

  def run_graph(t0, col_h, row_h, val_h, outs, nch, rp, ospm):
    nsl = pl.ds(0, nch)
    pltpu.sync_copy(row_h.at[sid], rowv.at[nsl])
    pltpu.sync_copy(val_h.at[sid], valv.at[nsl])
    src = t0
    for out_hbm in outs:
      layer(src, col_h, out_hbm, ospm, nch, rp)
      src = out_hbm

  rows0 = pl.ds(sid * UI_RP, UI_RP)
  pltpu.sync_copy(zeros_hbm.at[rows0], ospm.at[rows0])
  plsc.subcore_barrier()
  run_graph(ui_t0, ui_col, ui_row, ui_val, (cur1, cur2, cur3),
            UI_NCH, UI_RP, ospm)
  run_graph(uu_t0, uu_col, uu_row, uu_val, (ucur1, ucur2),
            UU_NCH, UU_RP, ospm)


def _combine_body(uE, iE, c1, c2, c3, u1, u2, ou, oi, ouu):
  s = c1[...] + c2[...] + c3[...]
  full = jnp.concatenate([s[0], s[1]], axis=1)
  ou[...] = uE[...] + full[:USER]
  oi[...] = iE[...] + full[USER:USER + ITEM]
  su = u1[...] + u2[...]
  fu = jnp.concatenate([su[0], su[1]], axis=1)
  ouu[...] = uE[...] + fu[:USER]


@jax.jit
def kernel(adj_indices, adj_values, uadj_indices, uadj_values, uEmbeds0,
           iEmbeds0):
  f32 = jnp.float32
  embeds = jnp.concatenate([uEmbeds0, iEmbeds0], axis=0)
  emb_p = jnp.pad(embeds, ((0, UI_R - USER - ITEM), (0, 0)))
  ui_t0 = jnp.stack([emb_p[:, :HALF], emb_p[:, HALF:]])
  ue_p = jnp.pad(uEmbeds0, ((0, UU_R - USER), (0, 0)))
  uu_t0 = jnp.stack([ue_p[:, :HALF], ue_p[:, HALF:]])

  ui_col = adj_indices[1].astype(jnp.int32).reshape(NS, UI_NCH, CH)
  ui_row = adj_indices[0].astype(jnp.int32).reshape(NS, UI_NCH, CH)
  ui_val = adj_values.reshape(NS, UI_NCH, CH)
  uu_col = uadj_indices[1].astype(jnp.int32).reshape(NS, UU_NCH, CH)
  uu_row = uadj_indices[0].astype(jnp.int32).reshape(NS, UU_NCH, CH)
  uu_val = uadj_values.reshape(NS, UU_NCH, CH)
  zeros_hbm = jnp.zeros((UI_R, HALF), f32)

  sds = jax.ShapeDtypeStruct
  sc = pl.kernel(
      _sc_kernel,
      out_type=[
          sds((NC, UI_R, HALF), f32),
          sds((NC, UI_R, HALF), f32),
          sds((NC, UI_R, HALF), f32),
          sds((NC, UU_R, HALF), f32),
          sds((NC, UU_R, HALF), f32),
      ],
      mesh=plsc.VectorSubcoreMesh(core_axis_name="c", subcore_axis_name="s"),
      compiler_params=pltpu.CompilerParams(use_tc_tiling_on_sc=False),
      scratch_types=[
          pltpu.VMEM((NBI, CH), jnp.int32),
          pltpu.VMEM((UI_NCH, CH), jnp.int32),
          pltpu.VMEM((UI_NCH, CH), f32),
          pltpu.VMEM((NB, CH, HALF), f32),
          pltpu.VMEM((NB, CH, HALF), f32),
          pltpu.VMEM_SHARED((UI_R, HALF), f32),
          pltpu.SemaphoreType.DMA((NB,)),
          pltpu.SemaphoreType.DMA((NB,)),
          pltpu.SemaphoreType.DMA((NBI,)),
      ],
  )
  cur1, cur2, cur3, ucur1, ucur2 = sc(
      ui_t0, uu_t0, ui_col, ui_row, ui_val, uu_col, uu_row, uu_val, zeros_hbm)

  ou, oi, ouu = pl.pallas_call(
      _combine_body,
      out_shape=[
          sds((USER, LATDIM), f32),
          sds((ITEM, LATDIM), f32),
          sds((USER, LATDIM), f32),
      ],
  )(uEmbeds0, iEmbeds0, cur1, cur2, cur3, ucur1, ucur2)
  return ou, oi, ouu


# speedup vs baseline: 6.5597x; 1.7979x over previous
import jax
import jax.numpy as jnp
from jax import lax
from jax.experimental import pallas as pl
from jax.experimental.pallas import tpu as pltpu
from jax.experimental.pallas import tpu_sc as plsc

USER = 5000
ITEM = 5000
LATDIM = 128
HALF = 64
N_EDGES = 320000
UU_EDGES = 160000
NC = 2
NS = 16
CH = 80
NB = 4
NBI = 3 * NB

UI_NCH = 252
UU_NCH = 128
UI_EP = NS * UI_NCH * CH
UU_EP = NS * UU_NCH * CH
UI_R = 10240
UU_R = 5120
UI_RP = UI_R // NS
UU_RP = UU_R // NS


def _sc_kernel(ui_t0, uu_t0, ui_col, ui_row, ui_val, uu_col, uu_row, uu_val,
               zeros_hbm,
               cur1, cur2, cur3, ucur1, ucur2,
               colring, rowv, valv, gbuf, sbuf, ospm, gsem, ssem, isem):
  cid = lax.axis_index("c")
  sid = lax.axis_index("s")

  def scale_chunk(j, b):
    @pl.loop(0, CH, step=16)
    def _(g):
      vv = valv[j, pl.ds(g, 16)]
      for e16 in range(16):
        bv = jnp.full((16,), vv[e16], dtype=jnp.float32)
        for c in range(HALF // 16):
          sl = (b, g + e16, pl.ds(c * 16, 16))
          sbuf[sl] = gbuf[sl] * bv

  def layer(t_src, col_h, out_hbm, ospm, nch, rp):
    src = t_src.at[cid]
    colh = col_h.at[sid]
    ngr = nch // NB

    def idx_dma(j):
      s = lax.rem(j, NBI)
      pltpu.async_copy(colh.at[j], colring.at[s], isem.at[s])

    def idx_wait(j):
      s = lax.rem(j, NBI)
      pltpu.make_async_copy(colh.at[j], colring.at[s], isem.at[s]).wait()

    def gather_start(j, b):
      s = lax.rem(j, NBI)
      pltpu.async_copy(src.at[colring.at[s]], gbuf.at[b], gsem.at[b])

    def gather_wait(j, b):
      s = lax.rem(j, NBI)
      pltpu.make_async_copy(src.at[colring.at[s]], gbuf.at[b],
                            gsem.at[b]).wait()

    @pl.loop(0, 2 * NB)
    def _(j):
      idx_dma(j)

    @pl.loop(0, NB)
    def _(j):
      idx_wait(j)
      b = lax.rem(j, NB)
      s = lax.rem(j, NBI)
      pltpu.async_copy(src.at[colring.at[s]], gbuf.at[b], gsem.at[b])

    @pl.loop(0, ngr)
    def _(jg):
      j0 = jg * NB
      for b in range(NB):
        j = j0 + b
        gather_wait(j, b)

        @pl.when(jg >= 1)
        def _():
          pltpu.make_async_copy(sbuf.at[b], ospm.at[rowv.at[j]],
                                ssem.at[b]).wait()

        scale_chunk(j, b)

        @pl.when(jg + 1 < ngr)
        def _():
          idx_wait(j + NB)
          gather_start(j + NB, b)

        @pl.when(jg + 2 < ngr)
        def _():
          idx_dma(j + 2 * NB)

        pltpu.async_copy(sbuf.at[b], ospm.at[rowv.at[j]], ssem.at[b], add=True)

    @pl.loop(nch - NB, nch)
    def _(j):
      b = lax.rem(j, NB)
      pltpu.make_async_copy(sbuf.at[b], ospm.at[rowv.at[j]], ssem.at[b]).wait()

    plsc.subcore_barrier()
    base = sid * rp
    rows = pl.ds(base, rp)
    pltpu.sync_copy(ospm.at[rows], out_hbm.at[cid].at[rows])
    pltpu.sync_copy(zeros_hbm.at[rows], ospm.at[rows])
    plsc.subcore_barrier()

  def run_graph(t0, col_h, row_h, val_h, outs, nch, rp, ospm):
    nsl = pl.ds(0, nch)
    pltpu.sync_copy(row_h.at[sid], rowv.at[nsl])
    pltpu.sync_copy(val_h.at[sid], valv.at[nsl])
    src = t0
    for out_hbm in outs:
      layer(src, col_h, out_hbm, ospm, nch, rp)
      src = out_hbm

  rows0 = pl.ds(sid * UI_RP, UI_RP)
  pltpu.sync_copy(zeros_hbm.at[rows0], ospm.at[rows0])
  plsc.subcore_barrier()
  run_graph(ui_t0, ui_col, ui_row, ui_val, (cur1, cur2, cur3),
            UI_NCH, UI_RP, ospm)
  run_graph(uu_t0, uu_col, uu_row, uu_val, (ucur1, ucur2),
            UU_NCH, UU_RP, ospm)


def _combine_body(uE, iE, c1, c2, c3, u1, u2, ou, oi, ouu):
  s = c1[...] + c2[...] + c3[...]
  full = jnp.concatenate([s[0], s[1]], axis=1)
  ou[...] = uE[...] + full[:USER]
  oi[...] = iE[...] + full[USER:USER + ITEM]
  su = u1[...] + u2[...]
  fu = jnp.concatenate([su[0], su[1]], axis=1)
  ouu[...] = uE[...] + fu[:USER]


@jax.jit
def kernel(adj_indices, adj_values, uadj_indices, uadj_values, uEmbeds0,
           iEmbeds0):
  f32 = jnp.float32
  embeds = jnp.concatenate([uEmbeds0, iEmbeds0], axis=0)
  emb_p = jnp.pad(embeds, ((0, UI_R - USER - ITEM), (0, 0)))
  ui_t0 = jnp.stack([emb_p[:, :HALF], emb_p[:, HALF:]])
  ue_p = jnp.pad(uEmbeds0, ((0, UU_R - USER), (0, 0)))
  uu_t0 = jnp.stack([ue_p[:, :HALF], ue_p[:, HALF:]])

  i32 = jnp.int32
  ui_pad = UI_EP - N_EDGES
  uu_pad = UU_EP - UU_EDGES
  ui_col = jnp.pad(adj_indices[1].astype(i32), (0, ui_pad)).reshape(
      NS, UI_NCH, CH)
  ui_row = jnp.pad(adj_indices[0].astype(i32), (0, ui_pad)).reshape(
      NS, UI_NCH, CH)
  ui_val = jnp.pad(adj_values, (0, ui_pad)).reshape(NS, UI_NCH, CH)
  uu_col = jnp.pad(uadj_indices[1].astype(i32), (0, uu_pad)).reshape(
      NS, UU_NCH, CH)
  uu_row = jnp.pad(uadj_indices[0].astype(i32), (0, uu_pad)).reshape(
      NS, UU_NCH, CH)
  uu_val = jnp.pad(uadj_values, (0, uu_pad)).reshape(NS, UU_NCH, CH)
  zeros_hbm = jnp.zeros((UI_R, HALF), f32)

  sds = jax.ShapeDtypeStruct
  sc = pl.kernel(
      _sc_kernel,
      out_type=[
          sds((NC, UI_R, HALF), f32),
          sds((NC, UI_R, HALF), f32),
          sds((NC, UI_R, HALF), f32),
          sds((NC, UU_R, HALF), f32),
          sds((NC, UU_R, HALF), f32),
      ],
      mesh=plsc.VectorSubcoreMesh(core_axis_name="c", subcore_axis_name="s"),
      compiler_params=pltpu.CompilerParams(use_tc_tiling_on_sc=False),
      scratch_types=[
          pltpu.VMEM((NBI, CH), jnp.int32),
          pltpu.VMEM((UI_NCH, CH), jnp.int32),
          pltpu.VMEM((UI_NCH, CH), f32),
          pltpu.VMEM((NB, CH, HALF), f32),
          pltpu.VMEM((NB, CH, HALF), f32),
          pltpu.VMEM_SHARED((UI_R, HALF), f32),
          pltpu.SemaphoreType.DMA((NB,)),
          pltpu.SemaphoreType.DMA((NB,)),
          pltpu.SemaphoreType.DMA((NBI,)),
      ],
  )
  cur1, cur2, cur3, ucur1, ucur2 = sc(
      ui_t0, uu_t0, ui_col, ui_row, ui_val, uu_col, uu_row, uu_val, zeros_hbm)

  ou, oi, ouu = pl.pallas_call(
      _combine_body,
      out_shape=[
          sds((USER, LATDIM), f32),
          sds((ITEM, LATDIM), f32),
          sds((USER, LATDIM), f32),
      ],
  )(uEmbeds0, iEmbeds0, cur1, cur2, cur3, ucur1, ucur2)
  return ou, oi, ouu
